# Initial kernel scaffold; baseline (speedup 1.0000x reference)
import functools
import jax
import jax.numpy as jnp
from jax import lax
from jax.experimental import pallas as pl
from jax.experimental.pallas import tpu as pltpu
from jax.experimental.pallas import tpu_sc as plsc

N = 10000
E = 320000
D = 128
NC = 2
NS = 16
NW = NC * NS
K = 128
NCH = -(-(E // NW) // K)
EPW = NCH * K
N_PAD = 10016
SLAB = N_PAD // NS

_mesh = plsc.VectorSubcoreMesh(
    core_axis_name="c", subcore_axis_name="s", num_cores=NC, num_subcores=NS)


def _zero_fill(ref, rows, width):
    def body(i, _):
        r = i // (width // 16)
        o = (i % (width // 16)) * 16
        ref[r, pl.ds(o, 16)] = jnp.zeros((16,), jnp.float32)
        return 0
    lax.fori_loop(0, rows * (width // 16), body, 0)


@functools.partial(
    pl.kernel,
    out_type=jax.ShapeDtypeStruct((NC, N_PAD, 16), jnp.float32),
    mesh=_mesh,
    scratch_types=[
        pltpu.VMEM_SHARED((N_PAD, 16), jnp.float32),
        pltpu.VMEM((NCH, K), jnp.int32),
        pltpu.VMEM((K, 16), jnp.float32),
        pltpu.VMEM((SLAB, 16), jnp.float32),
    ],
)
def _degree_kernel(coli_hbm, deg_out, acc, coli_v, ones_v, zer_v):
    c = lax.axis_index("c")
    s = lax.axis_index("s")
    w = c * NS + s
    base = s * SLAB

    pltpu.sync_copy(coli_hbm.at[w], coli_v)

    def fill_ones(i, _):
        ones_v[i, pl.ds(0, 16)] = jnp.ones((16,), jnp.float32)
        return 0
    lax.fori_loop(0, K, fill_ones, 0)
    _zero_fill(zer_v, SLAB, 16)
    pltpu.sync_copy(zer_v, acc.at[pl.ds(base, SLAB)])
    plsc.subcore_barrier()

    def chunk(j, _):
        pltpu.sync_copy(ones_v, acc.at[coli_v.at[j]], add=True)
        return 0
    lax.fori_loop(0, NCH, chunk, 0)

    plsc.subcore_barrier()
    pltpu.sync_copy(acc.at[pl.ds(base, SLAB)], deg_out.at[c, pl.ds(base, SLAB)])


@functools.partial(
    pl.kernel,
    out_type=jax.ShapeDtypeStruct((NC, N_PAD, D), jnp.float32),
    mesh=_mesh,
    scratch_types=[
        pltpu.VMEM_SHARED((N_PAD, D), jnp.float32),
        pltpu.VMEM((NCH, K), jnp.int32),
        pltpu.VMEM((NCH, K), jnp.int32),
        pltpu.VMEM((K, D), jnp.float32),
        pltpu.VMEM((K, D), jnp.float32),
        pltpu.VMEM((K, D), jnp.float32),
        pltpu.SemaphoreType.DMA,
        pltpu.SemaphoreType.DMA,
    ],
)
def _agg_kernel(src_hbm, rowi_hbm, coli_hbm, part_out,
                acc, rowi_v, coli_v, gb0, gb1, zer_v, sem0, sem1):
    c = lax.axis_index("c")
    s = lax.axis_index("s")
    w = c * NS + s
    base = s * SLAB

    pltpu.sync_copy(rowi_hbm.at[w], rowi_v)
    pltpu.sync_copy(coli_hbm.at[w], coli_v)

    _zero_fill(zer_v, K, D)
    for off in range(0, SLAB - K, K):
        pltpu.sync_copy(zer_v, acc.at[pl.ds(base + off, K)])
    rem = SLAB - (SLAB // K) * K
    pltpu.sync_copy(zer_v.at[pl.ds(0, rem)],
                    acc.at[pl.ds(base + (SLAB // K) * K, rem)])
    plsc.subcore_barrier()

    pltpu.async_copy(src_hbm.at[rowi_v.at[0]], gb0, sem0)

    def pair(i, _):
        j = 2 * i
        pltpu.make_async_copy(src_hbm.at[rowi_v.at[j]], gb0, sem0).wait()
        pltpu.async_copy(src_hbm.at[rowi_v.at[j + 1]], gb1, sem1)
        pltpu.sync_copy(gb0, acc.at[coli_v.at[j]], add=True)
        pltpu.make_async_copy(src_hbm.at[rowi_v.at[j + 1]], gb1, sem1).wait()
        pltpu.async_copy(src_hbm.at[rowi_v.at[j + 2]], gb0, sem0)
        pltpu.sync_copy(gb1, acc.at[coli_v.at[j + 1]], add=True)
        return 0
    lax.fori_loop(0, (NCH - 1) // 2, pair, 0)

    pltpu.make_async_copy(src_hbm.at[rowi_v.at[NCH - 1]], gb0, sem0).wait()
    pltpu.sync_copy(gb0, acc.at[coli_v.at[NCH - 1]], add=True)

    plsc.subcore_barrier()
    pltpu.sync_copy(acc.at[pl.ds(base, SLAB)],
                    part_out.at[c, pl.ds(base, SLAB)])


def _dis(deg0, deg1):
    d = deg0[0, :, 0:1] + deg1[0, :, 0:1] + 1.0
    return lax.rsqrt(d)


def _mm1_body(x_ref, w_ref, deg0_ref, deg1_ref, o_ref):
    h = jnp.dot(x_ref[...], w_ref[...], preferred_element_type=jnp.float32)
    o_ref[...] = h * _dis(deg0_ref, deg1_ref)


def _mm2_body(p0_ref, p1_ref, hw_ref, deg0_ref, deg1_ref, w_ref, c1_ref,
              o_ref):
    dis = _dis(deg0_ref, deg1_ref)
    a1 = dis * (p0_ref[0] + p1_ref[0] + hw_ref[...])
    h1 = jnp.maximum(a1 + c1_ref[...], 0.0)
    h2 = jnp.dot(h1, w_ref[...], preferred_element_type=jnp.float32)
    o_ref[...] = h2 * dis


def _fin_body(q0_ref, q1_ref, g_ref, deg0_ref, deg1_ref, b2_ref, o_ref):
    dis = _dis(deg0_ref, deg1_ref)
    o_ref[...] = dis * (q0_ref[0] + q1_ref[0] + g_ref[...]) + b2_ref[...]


_BR = 1000


def _row_spec():
    return pl.BlockSpec((_BR, D), lambda i: (i, 0))


def _part_spec(which):
    return pl.BlockSpec((1, _BR, D), lambda i, w=which: (w, i, 0))


def _deg_spec(which):
    return pl.BlockSpec((1, _BR, 16), lambda i, w=which: (w, i, 0))


def _vec_spec():
    return pl.BlockSpec((1, D), lambda i: (0, 0))


def kernel(x, edge_index, W1, b1, gamma1, beta1, W2, b2):
    B, S, C = x.shape
    x2d = x.reshape(N, D)
    row = edge_index[0].astype(jnp.int32)
    col = edge_index[1].astype(jnp.int32)
    pad = NW * EPW - E
    rowi = jnp.pad(row, (0, pad)).reshape(NW, NCH, K)
    coli = jnp.pad(col, (0, pad), constant_values=N).reshape(NW, NCH, K)

    sc = gamma1 * lax.rsqrt(jnp.float32(1.0 + 1e-5))
    W1p = W1 * sc[None, :]
    c1 = (b1 * sc + beta1).reshape(1, D)
    b2r = b2.reshape(1, D)

    deg = _degree_kernel(coli)

    grid = (N // _BR,)
    hw = pl.pallas_call(
        _mm1_body,
        grid=grid,
        in_specs=[_row_spec(), pl.BlockSpec((D, D), lambda i: (0, 0)),
                  _deg_spec(0), _deg_spec(1)],
        out_specs=_row_spec(),
        out_shape=jax.ShapeDtypeStruct((N, D), jnp.float32),
    )(x2d, W1p, deg, deg)

    p = _agg_kernel(hw, rowi, coli)

    g = pl.pallas_call(
        _mm2_body,
        grid=grid,
        in_specs=[_part_spec(0), _part_spec(1), _row_spec(),
                  _deg_spec(0), _deg_spec(1),
                  pl.BlockSpec((D, D), lambda i: (0, 0)), _vec_spec()],
        out_specs=_row_spec(),
        out_shape=jax.ShapeDtypeStruct((N, D), jnp.float32),
    )(p, p, hw, deg, deg, W2, c1)

    q = _agg_kernel(g, rowi, coli)

    out = pl.pallas_call(
        _fin_body,
        grid=grid,
        in_specs=[_part_spec(0), _part_spec(1), _row_spec(),
                  _deg_spec(0), _deg_spec(1), _vec_spec()],
        out_specs=_row_spec(),
        out_shape=jax.ShapeDtypeStruct((N, D), jnp.float32),
    )(q, q, g, deg, deg, b2r)

    return out.reshape(B, N, D)


# speedup vs baseline: 10.5885x; 10.5885x over previous
import functools
import jax
import jax.numpy as jnp
from jax import lax
from jax.experimental import pallas as pl
from jax.experimental.pallas import tpu as pltpu
from jax.experimental.pallas import tpu_sc as plsc

N = 10000
E = 320000
D = 128
NC = 2
NS = 16
NW = NC * NS
K = 128
NCH = -(-(E // NW) // K)
EPW = NCH * K
N_PAD = 10112
SLAB = N_PAD // NS

_mesh = plsc.VectorSubcoreMesh(
    core_axis_name="c", subcore_axis_name="s", num_cores=NC, num_subcores=NS)


def _fill(ref, rows, width, value):
    for r in range(rows):
        for j in range(width // 16):
            ref[r, pl.ds(16 * j, 16)] = jnp.full((16,), value, jnp.float32)


@functools.partial(
    pl.kernel,
    out_type=jax.ShapeDtypeStruct((NC, N_PAD, 16), jnp.float32),
    mesh=_mesh,
    scratch_types=[
        pltpu.VMEM_SHARED((N_PAD, 16), jnp.float32),
        pltpu.VMEM((K,), jnp.int32),
        pltpu.VMEM((K, 16), jnp.float32),
        pltpu.VMEM((8, 16), jnp.float32),
    ],
)
def _degree_kernel(coli_hbm, deg_out, acc, ci, ones_v, zer_v):
    c = lax.axis_index("c")
    s = lax.axis_index("s")
    w = c * NS + s
    base = s * SLAB

    _fill(ones_v, K, 16, 1.0)
    _fill(zer_v, 8, 16, 0.0)

    def zslab(i, _):
        pltpu.sync_copy(zer_v, acc.at[pl.ds(base + i * 8, 8)])
        return 0
    lax.fori_loop(0, SLAB // 8, zslab, 0)
    plsc.subcore_barrier()

    def chunk(j, _):
        pltpu.sync_copy(coli_hbm.at[w, j], ci)
        pltpu.sync_copy(ones_v, acc.at[ci], add=True)
        return 0
    lax.fori_loop(0, NCH, chunk, 0)

    plsc.subcore_barrier()
    pltpu.sync_copy(acc.at[pl.ds(base, SLAB)], deg_out.at[c, pl.ds(base, SLAB)])


@functools.partial(
    pl.kernel,
    out_type=jax.ShapeDtypeStruct((NC, N_PAD, D), jnp.float32),
    mesh=_mesh,
    scratch_types=[
        pltpu.VMEM_SHARED((N_PAD, D), jnp.float32),
        pltpu.VMEM((K,), jnp.int32),
        pltpu.VMEM((K,), jnp.int32),
        pltpu.VMEM((K, D), jnp.float32),
        pltpu.VMEM((8, D), jnp.float32),
        pltpu.SemaphoreType.DMA,
    ],
)
def _agg_kernel(src_hbm, rowi_hbm, coli_hbm, part_out,
                acc, ri, ci, gb0, zer_v, sem0):
    c = lax.axis_index("c")
    s = lax.axis_index("s")
    w = c * NS + s
    base = s * SLAB

    _fill(zer_v, 8, D, 0.0)
    def zslab(i, _):
        pltpu.sync_copy(zer_v, acc.at[pl.ds(base + i * 8, 8)])
        return 0
    lax.fori_loop(0, SLAB // 8, zslab, 0)
    plsc.subcore_barrier()

    def chunk(j, _):
        pltpu.sync_copy(rowi_hbm.at[w, j], ri)
        pltpu.sync_copy(coli_hbm.at[w, j], ci)
        pltpu.async_copy(src_hbm.at[ri], gb0, sem0).wait()
        pltpu.sync_copy(gb0, acc.at[ci], add=True)
        return 0
    lax.fori_loop(0, NCH, chunk, 0)

    plsc.subcore_barrier()
    pltpu.sync_copy(acc.at[pl.ds(base, SLAB)],
                    part_out.at[c, pl.ds(base, SLAB)])


def _dis(deg0, deg1):
    d = deg0[0, :, 0:1] + deg1[0, :, 0:1] + 1.0
    return lax.rsqrt(d)


def _mm1_body(x_ref, w_ref, deg0_ref, deg1_ref, o_ref):
    h = jnp.dot(x_ref[...], w_ref[...], preferred_element_type=jnp.float32)
    o_ref[...] = h * _dis(deg0_ref, deg1_ref)


def _mm2_body(p0_ref, p1_ref, hw_ref, deg0_ref, deg1_ref, w_ref, c1_ref,
              o_ref):
    dis = _dis(deg0_ref, deg1_ref)
    a1 = dis * (p0_ref[0] + p1_ref[0] + hw_ref[...])
    h1 = jnp.maximum(a1 + c1_ref[...], 0.0)
    h2 = jnp.dot(h1, w_ref[...], preferred_element_type=jnp.float32)
    o_ref[...] = h2 * dis


def _fin_body(q0_ref, q1_ref, g_ref, deg0_ref, deg1_ref, b2_ref, o_ref):
    dis = _dis(deg0_ref, deg1_ref)
    o_ref[...] = dis * (q0_ref[0] + q1_ref[0] + g_ref[...]) + b2_ref[...]


_BR = 1000


def _row_spec():
    return pl.BlockSpec((_BR, D), lambda i: (i, 0))


def _part_spec(which):
    return pl.BlockSpec((1, _BR, D), lambda i, w=which: (w, i, 0))


def _deg_spec(which):
    return pl.BlockSpec((1, _BR, 16), lambda i, w=which: (w, i, 0))


def _vec_spec():
    return pl.BlockSpec((1, D), lambda i: (0, 0))


def kernel(x, edge_index, W1, b1, gamma1, beta1, W2, b2):
    B, S, C = x.shape
    x2d = x.reshape(N, D)
    row = edge_index[0].astype(jnp.int32)
    col = edge_index[1].astype(jnp.int32)
    pad = NW * EPW - E
    rowi = jnp.pad(row, (0, pad)).reshape(NW, NCH, K)
    coli = jnp.pad(col, (0, pad), constant_values=N).reshape(NW, NCH, K)

    sc = gamma1 * lax.rsqrt(jnp.float32(1.0 + 1e-5))
    W1p = W1 * sc[None, :]
    c1 = (b1 * sc + beta1).reshape(1, D)
    b2r = b2.reshape(1, D)

    deg = _degree_kernel(coli)

    grid = (N // _BR,)
    hw = pl.pallas_call(
        _mm1_body,
        grid=grid,
        in_specs=[_row_spec(), pl.BlockSpec((D, D), lambda i: (0, 0)),
                  _deg_spec(0), _deg_spec(1)],
        out_specs=_row_spec(),
        out_shape=jax.ShapeDtypeStruct((N, D), jnp.float32),
    )(x2d, W1p, deg, deg)

    p = _agg_kernel(hw, rowi, coli)

    g = pl.pallas_call(
        _mm2_body,
        grid=grid,
        in_specs=[_part_spec(0), _part_spec(1), _row_spec(),
                  _deg_spec(0), _deg_spec(1),
                  pl.BlockSpec((D, D), lambda i: (0, 0)), _vec_spec()],
        out_specs=_row_spec(),
        out_shape=jax.ShapeDtypeStruct((N, D), jnp.float32),
    )(p, p, hw, deg, deg, W2, c1)

    q = _agg_kernel(g, rowi, coli)

    out = pl.pallas_call(
        _fin_body,
        grid=grid,
        in_specs=[_part_spec(0), _part_spec(1), _row_spec(),
                  _deg_spec(0), _deg_spec(1), _vec_spec()],
        out_specs=_row_spec(),
        out_shape=jax.ShapeDtypeStruct((N, D), jnp.float32),
    )(q, q, g, deg, deg, b2r)

    return out.reshape(B, N, D)
